# baseline (speedup 1.0000x reference)
import numpy as np
import jax
import jax.numpy as jnp
from jax.experimental import pallas as pl
from jax.experimental.pallas import tpu as pltpu

E = 768
H = 12
DH = 64
BS = 16
NG = 2
NW = 3
NR = 3
S = 4096
B = 2
NB = S // BS
KB = NG + NW + NR
ROWS = 512
NSC = S // ROWS


def _block_pattern():
    rng = np.random.default_rng(0)
    half = NW // 2
    offsets = np.arange(NW) - half
    rows = []
    for i in range(NB):
        g = np.arange(NG)
        w = np.clip(i + offsets, 0, NB - 1)
        r = rng.integers(0, NB, size=NR)
        rows.append(np.concatenate([g, w, r]))
    idx = np.stack(rows).astype(np.int32)
    dup = (idx[:, :, None] == idx[:, None, :]) & np.tril(
        np.ones((KB, KB), dtype=bool), -1)[None]
    is_dup = dup.any(-1)
    return idx, is_dup


_IDX_NP, _ISDUP_NP = _block_pattern()
_MASK_NP = np.repeat(
    np.where(_ISDUP_NP, np.float32(-1e9), np.float32(0.0)), BS, axis=1)
_MASK3_NP = np.broadcast_to(_MASK_NP[:, None, :], (NB, BS, KB * BS)).copy()


def _proj_kernel(q_ref, k_ref, v_ref, qw_ref, kw_ref, vw_ref, b_ref,
                 qo_ref, ko_ref, vo_ref):
    x_q = q_ref[:, 0, 0, :]
    x_k = k_ref[:, 0, 0, :]
    x_v = v_ref[:, 0, 0, :]
    bias = b_ref[...]
    rq = jnp.dot(x_q, qw_ref[...], preferred_element_type=jnp.float32) \
        + bias[0:1, 0:E]
    rk = jnp.dot(x_k, kw_ref[...], preferred_element_type=jnp.float32) \
        + bias[0:1, E:2 * E]
    rv = jnp.dot(x_v, vw_ref[...], preferred_element_type=jnp.float32) \
        + bias[0:1, 2 * E:3 * E]
    for h in range(H):
        sl = slice(h * DH, (h + 1) * DH)
        qo_ref[0, h] = rq[:, sl]
        ko_ref[0, h] = rk[:, sl]
        vo_ref[0, h] = rv[:, sl]


def _attn_kernel(idx_ref, q_ref, k_ref, v_ref, mask_ref, owt_ref, ob_ref,
                 out_ref, kg_ref, vg_ref, oh_ref):
    h = pl.program_id(1)

    @pl.when(h == 0)
    def _init():
        out_ref[:, 0, 0, :] = jnp.broadcast_to(ob_ref[...], (S, E))

    def body(n, _):
        for j in range(KB):
            src = idx_ref[n * KB + j] * BS
            kg_ref[pl.ds(j * BS, BS), :] = k_ref[0, 0, pl.ds(src, BS), :]
            vg_ref[pl.ds(j * BS, BS), :] = v_ref[0, 0, pl.ds(src, BS), :]
        q_n = q_ref[0, 0, pl.ds(n * BS, BS), :]
        scores = jax.lax.dot_general(
            q_n, kg_ref[...], (((1,), (1,)), ((), ())),
            preferred_element_type=jnp.float32)
        scores = scores + mask_ref[n]
        m = jnp.max(scores, axis=-1, keepdims=True)
        e = jnp.exp(scores - m)
        p = e / jnp.sum(e, axis=-1, keepdims=True)
        oh_ref[pl.ds(n * BS, BS), :] = jnp.dot(
            p, vg_ref[...], preferred_element_type=jnp.float32)
        return 0

    jax.lax.fori_loop(0, NB, body, 0)
    out_ref[:, 0, 0, :] += jnp.dot(
        oh_ref[...], owt_ref[...], preferred_element_type=jnp.float32)


def kernel(query, key, value, q_w, k_w, v_w, q_b, k_b, v_b, out_w, out_b):
    scale = 1.0 / np.sqrt(np.float32(DH))
    qwt = jnp.transpose(q_w, (1, 0, 2)).reshape(E, E) * scale
    kwt = jnp.transpose(k_w, (1, 0, 2)).reshape(E, E)
    vwt = jnp.transpose(v_w, (1, 0, 2)).reshape(E, E)
    bias = jnp.concatenate(
        [q_b.reshape(1, E) * scale, k_b.reshape(1, E), v_b.reshape(1, E)],
        axis=1)
    owt = out_w.T
    obr = out_b.reshape(1, E)

    q4 = query.reshape(S, B, 1, E)
    k4 = key.reshape(S, B, 1, E)
    v4 = value.reshape(S, B, 1, E)

    qkv = pl.pallas_call(
        _proj_kernel,
        grid=(B, NSC),
        in_specs=[
            pl.BlockSpec((ROWS, 1, 1, E), lambda b, s: (s, b, 0, 0)),
            pl.BlockSpec((ROWS, 1, 1, E), lambda b, s: (s, b, 0, 0)),
            pl.BlockSpec((ROWS, 1, 1, E), lambda b, s: (s, b, 0, 0)),
            pl.BlockSpec((E, E), lambda b, s: (0, 0)),
            pl.BlockSpec((E, E), lambda b, s: (0, 0)),
            pl.BlockSpec((E, E), lambda b, s: (0, 0)),
            pl.BlockSpec((1, 3 * E), lambda b, s: (0, 0)),
        ],
        out_specs=[
            pl.BlockSpec((1, H, ROWS, DH), lambda b, s: (b, 0, s, 0)),
            pl.BlockSpec((1, H, ROWS, DH), lambda b, s: (b, 0, s, 0)),
            pl.BlockSpec((1, H, ROWS, DH), lambda b, s: (b, 0, s, 0)),
        ],
        out_shape=[jax.ShapeDtypeStruct((B, H, S, DH), jnp.float32)] * 3,
    )(q4, k4, v4, qwt, kwt, vwt, bias)
    Q, K, V = qkv

    idx_flat = jnp.asarray(_IDX_NP.reshape(-1))
    mask3 = jnp.asarray(_MASK3_NP)

    p4 = pl.pallas_call(
        _attn_kernel,
        grid=(B, H),
        in_specs=[
            pl.BlockSpec(memory_space=pltpu.SMEM),
            pl.BlockSpec((1, 1, S, DH), lambda b, h: (b, h, 0, 0)),
            pl.BlockSpec((1, 1, S, DH), lambda b, h: (b, h, 0, 0)),
            pl.BlockSpec((1, 1, S, DH), lambda b, h: (b, h, 0, 0)),
            pl.BlockSpec((NB, BS, KB * BS), lambda b, h: (0, 0, 0)),
            pl.BlockSpec((DH, E), lambda b, h: (h, 0)),
            pl.BlockSpec((1, E), lambda b, h: (0, 0)),
        ],
        out_specs=pl.BlockSpec((S, 1, 1, E), lambda b, h: (0, b, 0, 0)),
        out_shape=jax.ShapeDtypeStruct((S, B, 1, E), jnp.float32),
        scratch_shapes=[
            pltpu.VMEM((KB * BS, DH), jnp.float32),
            pltpu.VMEM((KB * BS, DH), jnp.float32),
            pltpu.VMEM((S, DH), jnp.float32),
        ],
    )(idx_flat, Q, K, V, mask3, owt, obr)

    return p4.reshape(S, B, E)


# speedup vs baseline: 6.3360x; 6.3360x over previous
import numpy as np
import jax
import jax.numpy as jnp
from jax.experimental import pallas as pl
from jax.experimental.pallas import tpu as pltpu

E = 768
H = 12
DH = 64
BS = 16
NG = 2
NW = 3
NR = 3
S = 4096
B = 2
NB = S // BS
KB = NG + NW + NR
ROWS = 512
NSC = S // ROWS
G = 8
CH = NB // G
NU = NG + (G + 2) + NR * G
UC = NU * BS


def _block_pattern():
    rng = np.random.default_rng(0)
    half = NW // 2
    offsets = np.arange(NW) - half
    rows = []
    for i in range(NB):
        g = np.arange(NG)
        w = np.clip(i + offsets, 0, NB - 1)
        r = rng.integers(0, NB, size=NR)
        rows.append(np.concatenate([g, w, r]))
    idx = np.stack(rows).astype(np.int32)
    dup = (idx[:, :, None] == idx[:, None, :]) & np.tril(
        np.ones((KB, KB), dtype=bool), -1)[None]
    is_dup = dup.any(-1)
    return idx, is_dup


def _union_pattern(idx, is_dup):
    slots = np.zeros((CH, NU), np.int32)
    mask = np.full((CH, G * BS, UC), -1e9, np.float32)
    for c in range(CH):
        wb = np.clip(c * G - 1 + np.arange(G + 2), 0, NB - 1)
        slots[c] = np.concatenate(
            [np.arange(NG), wb, idx[c * G:(c + 1) * G, NG + NW:].reshape(-1)])
        for r in range(G):
            n = c * G + r
            for j in range(KB):
                if is_dup[n, j]:
                    continue
                v = idx[n, j]
                if j < NG:
                    u = j
                elif j < NG + NW:
                    u = NG + int(np.nonzero(wb == v)[0][0])
                else:
                    u = NG + (G + 2) + r * NR + (j - NG - NW)
                mask[c, r * BS:(r + 1) * BS, u * BS:(u + 1) * BS] = 0.0
    return slots.reshape(-1), mask


_IDX_NP, _ISDUP_NP = _block_pattern()
_SLOTS_NP, _MASK_NP = _union_pattern(_IDX_NP, _ISDUP_NP)


def _proj_kernel(q_ref, k_ref, v_ref, qw_ref, kw_ref, vw_ref, b_ref,
                 qo_ref, ko_ref, vo_ref):
    bias = b_ref[...]
    for x_ref, w_ref, o_ref, i in (
            (q_ref, qw_ref, qo_ref, 0),
            (k_ref, kw_ref, ko_ref, 1),
            (v_ref, vw_ref, vo_ref, 2)):
        x = x_ref[:, 0, 0, :]
        r = jnp.dot(x, w_ref[...], preferred_element_type=jnp.float32)
        o_ref[0] = (r + bias[0:1, i * E:(i + 1) * E]).astype(jnp.bfloat16)


def _attn_kernel(slots_ref, q_ref, k_ref, v_ref, mask_ref, owt_ref, ob_ref,
                 out_ref, kg_ref, vg_ref):
    c = pl.program_id(1)
    for u in range(NU):
        src = slots_ref[c * NU + u] * BS
        kg_ref[u * BS:(u + 1) * BS, :] = k_ref[0, pl.ds(src, BS), :]
        vg_ref[u * BS:(u + 1) * BS, :] = v_ref[0, pl.ds(src, BS), :]
    kgt = jnp.transpose(kg_ref[...], (1, 0))
    q = q_ref[0]
    msk = mask_ref[0]
    outs = []
    for h in range(H):
        sl = slice(h * DH, (h + 1) * DH)
        s = jnp.dot(q[:, sl], kgt[sl, :],
                    preferred_element_type=jnp.float32)
        e = jnp.exp(s + msk)
        p = (e / jnp.sum(e, axis=-1, keepdims=True)).astype(jnp.bfloat16)
        outs.append(jnp.dot(p, vg_ref[:, sl],
                            preferred_element_type=jnp.float32))
    oc = jnp.concatenate(outs, axis=1).astype(jnp.bfloat16)
    po = jnp.dot(oc, owt_ref[...], preferred_element_type=jnp.float32)
    out_ref[:, 0, 0, :] = po + ob_ref[...]


def kernel(query, key, value, q_w, k_w, v_w, q_b, k_b, v_b, out_w, out_b):
    scale = 1.0 / np.sqrt(np.float32(DH))
    qwt = jnp.transpose(q_w, (1, 0, 2)).reshape(E, E) * scale
    kwt = jnp.transpose(k_w, (1, 0, 2)).reshape(E, E)
    vwt = jnp.transpose(v_w, (1, 0, 2)).reshape(E, E)
    bias = jnp.concatenate(
        [q_b.reshape(1, E) * scale, k_b.reshape(1, E), v_b.reshape(1, E)],
        axis=1)
    owt = out_w.T.astype(jnp.bfloat16)
    obr = out_b.reshape(1, E)

    q4 = query.reshape(S, B, 1, E)
    k4 = key.reshape(S, B, 1, E)
    v4 = value.reshape(S, B, 1, E)

    Q, K, V = pl.pallas_call(
        _proj_kernel,
        grid=(B, NSC),
        in_specs=[
            pl.BlockSpec((ROWS, 1, 1, E), lambda b, s: (s, b, 0, 0)),
            pl.BlockSpec((ROWS, 1, 1, E), lambda b, s: (s, b, 0, 0)),
            pl.BlockSpec((ROWS, 1, 1, E), lambda b, s: (s, b, 0, 0)),
            pl.BlockSpec((E, E), lambda b, s: (0, 0)),
            pl.BlockSpec((E, E), lambda b, s: (0, 0)),
            pl.BlockSpec((E, E), lambda b, s: (0, 0)),
            pl.BlockSpec((1, 3 * E), lambda b, s: (0, 0)),
        ],
        out_specs=[
            pl.BlockSpec((1, ROWS, E), lambda b, s: (b, s, 0)),
            pl.BlockSpec((1, ROWS, E), lambda b, s: (b, s, 0)),
            pl.BlockSpec((1, ROWS, E), lambda b, s: (b, s, 0)),
        ],
        out_shape=[jax.ShapeDtypeStruct((B, S, E), jnp.bfloat16)] * 3,
    )(q4, k4, v4, qwt, kwt, vwt, bias)

    slots = jnp.asarray(_SLOTS_NP)
    mask = jnp.asarray(_MASK_NP)

    p4 = pl.pallas_call(
        _attn_kernel,
        grid=(B, CH),
        in_specs=[
            pl.BlockSpec(memory_space=pltpu.SMEM),
            pl.BlockSpec((1, G * BS, E), lambda b, c: (b, c, 0)),
            pl.BlockSpec((1, S, E), lambda b, c: (b, 0, 0)),
            pl.BlockSpec((1, S, E), lambda b, c: (b, 0, 0)),
            pl.BlockSpec((1, G * BS, UC), lambda b, c: (c, 0, 0)),
            pl.BlockSpec((E, E), lambda b, c: (0, 0)),
            pl.BlockSpec((1, E), lambda b, c: (0, 0)),
        ],
        out_specs=pl.BlockSpec((G * BS, 1, 1, E), lambda b, c: (c, b, 0, 0)),
        out_shape=jax.ShapeDtypeStruct((S, B, 1, E), jnp.float32),
        scratch_shapes=[
            pltpu.VMEM((UC, E), jnp.bfloat16),
            pltpu.VMEM((UC, E), jnp.bfloat16),
        ],
    )(slots, Q, K, V, mask, owt, obr)

    return p4.reshape(S, B, E)


# speedup vs baseline: 8.0680x; 1.2734x over previous
import numpy as np
import jax
import jax.numpy as jnp
from jax.experimental import pallas as pl
from jax.experimental.pallas import tpu as pltpu

E = 768
H = 12
DH = 64
BS = 16
NG = 2
NW = 3
NR = 3
S = 4096
B = 2
NB = S // BS
KB = NG + NW + NR
ROWS = 512
NSC = S // ROWS
G = 8
CH = NB // G
NU = NG + (G + 2) + NR * G
UC = NU * BS


def _block_pattern():
    rng = np.random.default_rng(0)
    half = NW // 2
    offsets = np.arange(NW) - half
    rows = []
    for i in range(NB):
        g = np.arange(NG)
        w = np.clip(i + offsets, 0, NB - 1)
        r = rng.integers(0, NB, size=NR)
        rows.append(np.concatenate([g, w, r]))
    idx = np.stack(rows).astype(np.int32)
    dup = (idx[:, :, None] == idx[:, None, :]) & np.tril(
        np.ones((KB, KB), dtype=bool), -1)[None]
    is_dup = dup.any(-1)
    return idx, is_dup


def _union_pattern(idx, is_dup):
    slots = np.zeros((CH, NU), np.int32)
    mask = np.full((CH, G * BS, UC), -1e9, np.float32)
    for c in range(CH):
        wb = np.clip(c * G - 1 + np.arange(G + 2), 0, NB - 1)
        slots[c] = np.concatenate(
            [np.arange(NG), wb, idx[c * G:(c + 1) * G, NG + NW:].reshape(-1)])
        for r in range(G):
            n = c * G + r
            for j in range(KB):
                if is_dup[n, j]:
                    continue
                v = idx[n, j]
                if j < NG:
                    u = j
                elif j < NG + NW:
                    u = NG + int(np.nonzero(wb == v)[0][0])
                else:
                    u = NG + (G + 2) + r * NR + (j - NG - NW)
                mask[c, r * BS:(r + 1) * BS, u * BS:(u + 1) * BS] = 0.0
    return slots.reshape(-1), mask


_IDX_NP, _ISDUP_NP = _block_pattern()
_SLOTS_NP, _MASK_NP = _union_pattern(_IDX_NP, _ISDUP_NP)


def _proj_kernel(q_ref, k_ref, v_ref, qw_ref, kw_ref, vw_ref, b_ref,
                 qo_ref, ko_ref, vo_ref):
    bias = b_ref[...]
    for x_ref, w_ref, o_ref, i in (
            (q_ref, qw_ref, qo_ref, 0),
            (k_ref, kw_ref, ko_ref, 1),
            (v_ref, vw_ref, vo_ref, 2)):
        x = x_ref[:, 0, 0, :]
        r = jnp.dot(x, w_ref[...], preferred_element_type=jnp.float32)
        r = r + bias[0:1, i * E:(i + 1) * E]
        o_ref[0] = r.astype(o_ref.dtype)


def _attn_kernel(slots_ref, q_ref, k_ref, v_ref, mask_ref, owt_ref, ob_ref,
                 out_ref, kg_ref, vg_ref):
    c = pl.program_id(1)
    for u in range(NU):
        src = slots_ref[c * NU + u] * BS
        kg_ref[u * BS:(u + 1) * BS, :] = k_ref[0, pl.ds(src, BS), :]
        vg_ref[u * BS:(u + 1) * BS, :] = v_ref[0, pl.ds(src, BS), :]
    kgt = jnp.transpose(kg_ref[...], (1, 0))
    q = q_ref[0]
    msk = mask_ref[0]
    avs = []
    recips = []
    for h in range(H):
        sl = slice(h * DH, (h + 1) * DH)
        s = jnp.dot(q[:, sl], kgt[sl, :],
                    preferred_element_type=jnp.float32)
        e = jnp.exp(s + msk)
        avs.append(jnp.dot(e, vg_ref[:, sl],
                           preferred_element_type=jnp.float32))
        recips.append(1.0 / jnp.sum(e, axis=-1, keepdims=True))
    oc = jnp.concatenate(
        [av * r for av, r in zip(avs, recips)], axis=1)
    po = jnp.dot(oc.astype(jnp.bfloat16), owt_ref[...],
                 preferred_element_type=jnp.float32)
    out_ref[:, 0, 0, :] = po + ob_ref[...]


def kernel(query, key, value, q_w, k_w, v_w, q_b, k_b, v_b, out_w, out_b):
    scale = 1.0 / np.sqrt(np.float32(DH))
    qwt = jnp.transpose(q_w, (1, 0, 2)).reshape(E, E) * scale
    kwt = jnp.transpose(k_w, (1, 0, 2)).reshape(E, E)
    vwt = jnp.transpose(v_w, (1, 0, 2)).reshape(E, E)
    bias = jnp.concatenate(
        [q_b.reshape(1, E) * scale, k_b.reshape(1, E), v_b.reshape(1, E)],
        axis=1)
    owt = out_w.T.astype(jnp.bfloat16)
    obr = out_b.reshape(1, E)

    q4 = query.reshape(S, B, 1, E)
    k4 = key.reshape(S, B, 1, E)
    v4 = value.reshape(S, B, 1, E)

    Q, K, V = pl.pallas_call(
        _proj_kernel,
        grid=(B, NSC),
        in_specs=[
            pl.BlockSpec((ROWS, 1, 1, E), lambda b, s: (s, b, 0, 0)),
            pl.BlockSpec((ROWS, 1, 1, E), lambda b, s: (s, b, 0, 0)),
            pl.BlockSpec((ROWS, 1, 1, E), lambda b, s: (s, b, 0, 0)),
            pl.BlockSpec((E, E), lambda b, s: (0, 0)),
            pl.BlockSpec((E, E), lambda b, s: (0, 0)),
            pl.BlockSpec((E, E), lambda b, s: (0, 0)),
            pl.BlockSpec((1, 3 * E), lambda b, s: (0, 0)),
        ],
        out_specs=[
            pl.BlockSpec((1, ROWS, E), lambda b, s: (b, s, 0)),
            pl.BlockSpec((1, ROWS, E), lambda b, s: (b, s, 0)),
            pl.BlockSpec((1, ROWS, E), lambda b, s: (b, s, 0)),
        ],
        out_shape=[jax.ShapeDtypeStruct((B, S, E), jnp.bfloat16),
                   jax.ShapeDtypeStruct((B, S, E), jnp.bfloat16),
                   jax.ShapeDtypeStruct((B, S, E), jnp.float32)],
    )(q4, k4, v4, qwt, kwt, vwt, bias)

    slots = jnp.asarray(_SLOTS_NP)
    mask = jnp.asarray(_MASK_NP)

    p4 = pl.pallas_call(
        _attn_kernel,
        grid=(B, CH),
        in_specs=[
            pl.BlockSpec(memory_space=pltpu.SMEM),
            pl.BlockSpec((1, G * BS, E), lambda b, c: (b, c, 0)),
            pl.BlockSpec((1, S, E), lambda b, c: (b, 0, 0)),
            pl.BlockSpec((1, S, E), lambda b, c: (b, 0, 0)),
            pl.BlockSpec((1, G * BS, UC), lambda b, c: (c, 0, 0)),
            pl.BlockSpec((E, E), lambda b, c: (0, 0)),
            pl.BlockSpec((1, E), lambda b, c: (0, 0)),
        ],
        out_specs=pl.BlockSpec((G * BS, 1, 1, E), lambda b, c: (c, b, 0, 0)),
        out_shape=jax.ShapeDtypeStruct((S, B, 1, E), jnp.float32),
        scratch_shapes=[
            pltpu.VMEM((UC, E), jnp.bfloat16),
            pltpu.VMEM((UC, E), jnp.float32),
        ],
    )(slots, Q, K, V, mask, owt, obr)

    return p4.reshape(S, B, E)


# speedup vs baseline: 8.2924x; 1.0278x over previous
import numpy as np
import jax
import jax.numpy as jnp
from jax.experimental import pallas as pl
from jax.experimental.pallas import tpu as pltpu

E = 768
H = 12
DH = 64
BS = 16
NG = 2
NW = 3
NR = 3
S = 4096
B = 2
NB = S // BS
KB = NG + NW + NR
ROWS = 1024
NSC = S // ROWS
G = 8
CH = NB // G
NU = NG + (G + 2) + NR * G
UC = NU * BS


def _block_pattern():
    rng = np.random.default_rng(0)
    half = NW // 2
    offsets = np.arange(NW) - half
    rows = []
    for i in range(NB):
        g = np.arange(NG)
        w = np.clip(i + offsets, 0, NB - 1)
        r = rng.integers(0, NB, size=NR)
        rows.append(np.concatenate([g, w, r]))
    idx = np.stack(rows).astype(np.int32)
    dup = (idx[:, :, None] == idx[:, None, :]) & np.tril(
        np.ones((KB, KB), dtype=bool), -1)[None]
    is_dup = dup.any(-1)
    return idx, is_dup


def _union_pattern(idx, is_dup):
    slots = np.zeros((CH, NU), np.int32)
    mask = np.full((CH, G * BS, UC), -1e9, np.float32)
    for c in range(CH):
        wb = np.clip(c * G - 1 + np.arange(G + 2), 0, NB - 1)
        slots[c] = np.concatenate(
            [np.arange(NG), wb, idx[c * G:(c + 1) * G, NG + NW:].reshape(-1)])
        for r in range(G):
            n = c * G + r
            for j in range(KB):
                if is_dup[n, j]:
                    continue
                v = idx[n, j]
                if j < NG:
                    u = j
                elif j < NG + NW:
                    u = NG + int(np.nonzero(wb == v)[0][0])
                else:
                    u = NG + (G + 2) + r * NR + (j - NG - NW)
                mask[c, r * BS:(r + 1) * BS, u * BS:(u + 1) * BS] = 0.0
    return slots.reshape(-1), mask


_IDX_NP, _ISDUP_NP = _block_pattern()
_SLOTS_NP, _MASK_NP = _union_pattern(_IDX_NP, _ISDUP_NP)


def _proj_kernel(q_ref, k_ref, v_ref, qw_ref, kw_ref, vw_ref, b_ref,
                 qo_ref, ko_ref, vo_ref):
    bias = b_ref[...]
    for x_ref, w_ref, o_ref, i in (
            (q_ref, qw_ref, qo_ref, 0),
            (k_ref, kw_ref, ko_ref, 1),
            (v_ref, vw_ref, vo_ref, 2)):
        x = x_ref[:, 0, 0, :]
        r = jnp.dot(x, w_ref[...], preferred_element_type=jnp.float32)
        r = r + bias[0:1, i * E:(i + 1) * E]
        o_ref[0] = r.astype(o_ref.dtype)


def _attn_kernel(slots_ref, q_ref, k_ref, v_ref, mask_ref, owt_ref, ob_ref,
                 out_ref, kg_ref, vg_ref):
    c = pl.program_id(1)
    for u in range(NU):
        src = slots_ref[c * NU + u] * BS
        kg_ref[u * BS:(u + 1) * BS, :] = k_ref[0, pl.ds(src, BS), :]
        vg_ref[u * BS:(u + 1) * BS, :] = v_ref[0, pl.ds(src, BS), :]
    kgt = jnp.transpose(kg_ref[...], (1, 0))
    q = q_ref[0]
    msk = mask_ref[0].astype(jnp.float32)
    avs = []
    recips = []
    for h in range(H):
        sl = slice(h * DH, (h + 1) * DH)
        s = jnp.dot(q[:, sl], kgt[sl, :],
                    preferred_element_type=jnp.float32)
        e = jnp.exp(s + msk)
        avs.append(jnp.dot(e.astype(jnp.bfloat16), vg_ref[:, sl],
                           preferred_element_type=jnp.float32))
        recips.append(1.0 / jnp.sum(e, axis=-1, keepdims=True))
    oc = jnp.concatenate(
        [av * r for av, r in zip(avs, recips)], axis=1)
    po = jnp.dot(oc.astype(jnp.bfloat16), owt_ref[...],
                 preferred_element_type=jnp.float32)
    out_ref[:, 0, 0, :] = po + ob_ref[...]


def kernel(query, key, value, q_w, k_w, v_w, q_b, k_b, v_b, out_w, out_b):
    scale = 1.0 / np.sqrt(np.float32(DH))
    qwt = jnp.transpose(q_w, (1, 0, 2)).reshape(E, E) * scale
    kwt = jnp.transpose(k_w, (1, 0, 2)).reshape(E, E)
    vwt = jnp.transpose(v_w, (1, 0, 2)).reshape(E, E)
    bias = jnp.concatenate(
        [q_b.reshape(1, E) * scale, k_b.reshape(1, E), v_b.reshape(1, E)],
        axis=1)
    owt = out_w.T.astype(jnp.bfloat16)
    obr = out_b.reshape(1, E)

    q4 = query.reshape(S, B, 1, E)
    k4 = key.reshape(S, B, 1, E)
    v4 = value.reshape(S, B, 1, E)

    Q, K, V = pl.pallas_call(
        _proj_kernel,
        grid=(B, NSC),
        in_specs=[
            pl.BlockSpec((ROWS, 1, 1, E), lambda b, s: (s, b, 0, 0)),
            pl.BlockSpec((ROWS, 1, 1, E), lambda b, s: (s, b, 0, 0)),
            pl.BlockSpec((ROWS, 1, 1, E), lambda b, s: (s, b, 0, 0)),
            pl.BlockSpec((E, E), lambda b, s: (0, 0)),
            pl.BlockSpec((E, E), lambda b, s: (0, 0)),
            pl.BlockSpec((E, E), lambda b, s: (0, 0)),
            pl.BlockSpec((1, 3 * E), lambda b, s: (0, 0)),
        ],
        out_specs=[
            pl.BlockSpec((1, ROWS, E), lambda b, s: (b, s, 0)),
            pl.BlockSpec((1, ROWS, E), lambda b, s: (b, s, 0)),
            pl.BlockSpec((1, ROWS, E), lambda b, s: (b, s, 0)),
        ],
        out_shape=[jax.ShapeDtypeStruct((B, S, E), jnp.bfloat16)] * 3,
    )(q4, k4, v4, qwt, kwt, vwt, bias)

    slots = jnp.asarray(_SLOTS_NP)
    mask = jnp.asarray(_MASK_NP).astype(jnp.bfloat16)

    p4 = pl.pallas_call(
        _attn_kernel,
        grid=(B, CH),
        in_specs=[
            pl.BlockSpec(memory_space=pltpu.SMEM),
            pl.BlockSpec((1, G * BS, E), lambda b, c: (b, c, 0)),
            pl.BlockSpec((1, S, E), lambda b, c: (b, 0, 0)),
            pl.BlockSpec((1, S, E), lambda b, c: (b, 0, 0)),
            pl.BlockSpec((1, G * BS, UC), lambda b, c: (c, 0, 0)),
            pl.BlockSpec((E, E), lambda b, c: (0, 0)),
            pl.BlockSpec((1, E), lambda b, c: (0, 0)),
        ],
        out_specs=pl.BlockSpec((G * BS, 1, 1, E), lambda b, c: (c, b, 0, 0)),
        out_shape=jax.ShapeDtypeStruct((S, B, 1, E), jnp.float32),
        scratch_shapes=[
            pltpu.VMEM((UC, E), jnp.bfloat16),
            pltpu.VMEM((UC, E), jnp.bfloat16),
        ],
    )(slots, Q, K, V, mask, owt, obr)

    return p4.reshape(S, B, E)


# speedup vs baseline: 8.4433x; 1.0182x over previous
import numpy as np
import jax
import jax.numpy as jnp
from jax.experimental import pallas as pl
from jax.experimental.pallas import tpu as pltpu

E = 768
H = 12
DH = 64
BS = 16
NG = 2
NW = 3
NR = 3
S = 4096
B = 2
NB = S // BS
KB = NG + NW + NR
ROWS = 1024
NSC = S // ROWS
G = 8
CH = NB // G
NU = NG + (G + 2) + NR * G
UC = NU * BS


def _block_pattern():
    rng = np.random.default_rng(0)
    half = NW // 2
    offsets = np.arange(NW) - half
    rows = []
    for i in range(NB):
        g = np.arange(NG)
        w = np.clip(i + offsets, 0, NB - 1)
        r = rng.integers(0, NB, size=NR)
        rows.append(np.concatenate([g, w, r]))
    idx = np.stack(rows).astype(np.int32)
    dup = (idx[:, :, None] == idx[:, None, :]) & np.tril(
        np.ones((KB, KB), dtype=bool), -1)[None]
    is_dup = dup.any(-1)
    return idx, is_dup


def _union_pattern(idx, is_dup):
    slots = np.zeros((CH, NU), np.int32)
    mask = np.full((CH, G * BS, UC), -1e9, np.float32)
    for c in range(CH):
        wb = np.clip(c * G - 1 + np.arange(G + 2), 0, NB - 1)
        slots[c] = np.concatenate(
            [np.arange(NG), wb, idx[c * G:(c + 1) * G, NG + NW:].reshape(-1)])
        for r in range(G):
            n = c * G + r
            for j in range(KB):
                if is_dup[n, j]:
                    continue
                v = idx[n, j]
                if j < NG:
                    u = j
                elif j < NG + NW:
                    u = NG + int(np.nonzero(wb == v)[0][0])
                else:
                    u = NG + (G + 2) + r * NR + (j - NG - NW)
                mask[c, r * BS:(r + 1) * BS, u * BS:(u + 1) * BS] = 0.0
    return slots.reshape(-1), mask


_IDX_NP, _ISDUP_NP = _block_pattern()
_SLOTS_NP, _MASK_NP = _union_pattern(_IDX_NP, _ISDUP_NP)


def _proj_kernel(q_ref, k_ref, v_ref, qw_ref, kw_ref, vw_ref, b_ref,
                 qo_ref, ko_ref, vo_ref):
    bias = b_ref[...]
    for x_ref, w_ref, o_ref, i in (
            (q_ref, qw_ref, qo_ref, 0),
            (k_ref, kw_ref, ko_ref, 1),
            (v_ref, vw_ref, vo_ref, 2)):
        x = x_ref[:, 0, 0, :]
        r = jnp.dot(x, w_ref[...], preferred_element_type=jnp.float32)
        r = r + bias[0:1, i * E:(i + 1) * E]
        o_ref[0] = r.astype(o_ref.dtype)


def _attn_kernel(slots_ref, q_ref, k_ref, v_ref, mask_ref, owt_ref, ob_ref,
                 out_ref, kg_ref, vg_ref):
    c = pl.program_id(0)
    msk = mask_ref[0].astype(jnp.float32)
    for b in range(B):
        for u in range(NU):
            src = slots_ref[c * NU + u] * BS
            kg_ref[b, u * BS:(u + 1) * BS, :] = k_ref[b, pl.ds(src, BS), :]
            vg_ref[b, u * BS:(u + 1) * BS, :] = v_ref[b, pl.ds(src, BS), :]
    for b in range(B):
        kgt = jnp.transpose(kg_ref[b], (1, 0))
        q = q_ref[b]
        avs = []
        recips = []
        for h in range(H):
            sl = slice(h * DH, (h + 1) * DH)
            s = jnp.dot(q[:, sl], kgt[sl, :],
                        preferred_element_type=jnp.float32)
            e = jnp.exp(s + msk)
            avs.append(jnp.dot(e.astype(jnp.bfloat16), vg_ref[b, :, sl],
                               preferred_element_type=jnp.float32))
            recips.append(1.0 / jnp.sum(e, axis=-1, keepdims=True))
        oc = jnp.concatenate(
            [av * r for av, r in zip(avs, recips)], axis=1)
        po = jnp.dot(oc.astype(jnp.bfloat16), owt_ref[...],
                     preferred_element_type=jnp.float32)
        out_ref[:, b, 0, :] = po + ob_ref[...]


def kernel(query, key, value, q_w, k_w, v_w, q_b, k_b, v_b, out_w, out_b):
    scale = 1.0 / np.sqrt(np.float32(DH))
    qwt = jnp.transpose(q_w, (1, 0, 2)).reshape(E, E) * scale
    kwt = jnp.transpose(k_w, (1, 0, 2)).reshape(E, E)
    vwt = jnp.transpose(v_w, (1, 0, 2)).reshape(E, E)
    bias = jnp.concatenate(
        [q_b.reshape(1, E) * scale, k_b.reshape(1, E), v_b.reshape(1, E)],
        axis=1)
    owt = out_w.T.astype(jnp.bfloat16)
    obr = out_b.reshape(1, E)

    q4 = query.reshape(S, B, 1, E)
    k4 = key.reshape(S, B, 1, E)
    v4 = value.reshape(S, B, 1, E)

    Q, K, V = pl.pallas_call(
        _proj_kernel,
        grid=(B, NSC),
        in_specs=[
            pl.BlockSpec((ROWS, 1, 1, E), lambda b, s: (s, b, 0, 0)),
            pl.BlockSpec((ROWS, 1, 1, E), lambda b, s: (s, b, 0, 0)),
            pl.BlockSpec((ROWS, 1, 1, E), lambda b, s: (s, b, 0, 0)),
            pl.BlockSpec((E, E), lambda b, s: (0, 0)),
            pl.BlockSpec((E, E), lambda b, s: (0, 0)),
            pl.BlockSpec((E, E), lambda b, s: (0, 0)),
            pl.BlockSpec((1, 3 * E), lambda b, s: (0, 0)),
        ],
        out_specs=[
            pl.BlockSpec((1, ROWS, E), lambda b, s: (b, s, 0)),
            pl.BlockSpec((1, ROWS, E), lambda b, s: (b, s, 0)),
            pl.BlockSpec((1, ROWS, E), lambda b, s: (b, s, 0)),
        ],
        out_shape=[jax.ShapeDtypeStruct((B, S, E), jnp.bfloat16)] * 3,
    )(q4, k4, v4, qwt, kwt, vwt, bias)

    slots = jnp.asarray(_SLOTS_NP)
    mask = jnp.asarray(_MASK_NP).astype(jnp.bfloat16)

    p4 = pl.pallas_call(
        _attn_kernel,
        grid=(CH,),
        in_specs=[
            pl.BlockSpec(memory_space=pltpu.SMEM),
            pl.BlockSpec((B, G * BS, E), lambda c: (0, c, 0)),
            pl.BlockSpec((B, S, E), lambda c: (0, 0, 0)),
            pl.BlockSpec((B, S, E), lambda c: (0, 0, 0)),
            pl.BlockSpec((1, G * BS, UC), lambda c: (c, 0, 0)),
            pl.BlockSpec((E, E), lambda c: (0, 0)),
            pl.BlockSpec((1, E), lambda c: (0, 0)),
        ],
        out_specs=pl.BlockSpec((G * BS, B, 1, E), lambda c: (c, 0, 0, 0)),
        out_shape=jax.ShapeDtypeStruct((S, B, 1, E), jnp.float32),
        scratch_shapes=[
            pltpu.VMEM((B, UC, E), jnp.bfloat16),
            pltpu.VMEM((B, UC, E), jnp.bfloat16),
        ],
    )(slots, Q, K, V, mask, owt, obr)

    return p4.reshape(S, B, E)
